# Initial kernel scaffold; baseline (speedup 1.0000x reference)
import jax
import jax.numpy as jnp
from jax.experimental import pallas as pl


def kernel(x, edge_index, batch, W0, b0, W1, b1, W2, b2, g0, be0, g1, be1):
    raise NotImplementedError("write your pallas kernel here")


# speedup vs baseline: 15.1883x; 15.1883x over previous
import functools

import jax
import jax.numpy as jnp
from jax import lax
from jax.experimental import pallas as pl
from jax.experimental.pallas import tpu as pltpu
from jax.experimental.pallas import tpu_sc as plsc

N = 10000
NP = 10240
E = 320000
EP = 323584
NCHUNK = 79
CL = 128
NC, NS = 2, 16
RPT = NP // NS
IN_CH, HID, OUT_CH, NG = 128, 64, 128, 16

_MESH = plsc.VectorSubcoreMesh(
    core_axis_name="c", subcore_axis_name="s", num_cores=NC, num_subcores=NS)
_SC_PARAMS = pltpu.CompilerParams(use_tc_tiling_on_sc=False)


@functools.partial(
    pl.kernel,
    out_type=jax.ShapeDtypeStruct((NC, NP), jnp.float32),
    mesh=_MESH,
    compiler_params=_SC_PARAMS,
    scratch_types=[
        pltpu.VMEM((NCHUNK, CL), jnp.int32),
        pltpu.VMEM((RPT,), jnp.float32),
        pltpu.VMEM((CL,), jnp.float32),
        pltpu.VMEM_SHARED((NP,), jnp.float32),
    ],
)
def _deg_kernel(dst_hbm, out_hbm, dst_v, zb, ones_v, acc):
    c = lax.axis_index("c")
    s = lax.axis_index("s")
    w = c * NS + s

    def zb_body(i, _):
        zb[pl.ds(i * 16, 16)] = jnp.zeros((16,), jnp.float32)
        return 0
    lax.fori_loop(0, RPT // 16, zb_body, 0)

    def ones_body(i, _):
        ones_v[pl.ds(i * 16, 16)] = jnp.ones((16,), jnp.float32)
        return 0
    lax.fori_loop(0, CL // 16, ones_body, 0)

    pltpu.sync_copy(zb, acc.at[pl.ds(s * RPT, RPT)])
    pltpu.sync_copy(dst_hbm.at[w], dst_v)
    plsc.subcore_barrier()

    def body(j, _):
        pltpu.sync_copy(ones_v, acc.at[dst_v.at[j]], add=True)
        return 0
    lax.fori_loop(0, NCHUNK, body, 0)

    plsc.subcore_barrier()
    pltpu.sync_copy(acc.at[pl.ds(s * RPT, RPT)],
                    out_hbm.at[c, pl.ds(s * RPT, RPT)])


def _make_agg(F):
    @functools.partial(
        pl.kernel,
        out_type=jax.ShapeDtypeStruct((NC, NP, F), jnp.float32),
        mesh=_MESH,
        compiler_params=_SC_PARAMS,
        scratch_types=[
            pltpu.VMEM((NCHUNK, CL), jnp.int32),
            pltpu.VMEM((NCHUNK, CL), jnp.int32),
            pltpu.VMEM((CL, F), jnp.float32),
            pltpu.VMEM_SHARED((NP, F), jnp.float32),
            pltpu.SemaphoreType.DMA,
        ],
    )
    def _agg(table_hbm, src_hbm, dst_hbm, out_hbm, src_v, dst_v, rows_v,
             acc, sem):
        c = lax.axis_index("c")
        s = lax.axis_index("s")
        w = c * NS + s

        def zr(i, _):
            def zc(j, __):
                rows_v[i, pl.ds(j * 16, 16)] = jnp.zeros((16,), jnp.float32)
                return 0
            return lax.fori_loop(0, F // 16, zc, 0)
        lax.fori_loop(0, CL, zr, 0)

        def zacc(k, _):
            pltpu.sync_copy(rows_v, acc.at[pl.ds(s * RPT + k * CL, CL)])
            return 0
        lax.fori_loop(0, RPT // CL, zacc, 0)

        pltpu.sync_copy(src_hbm.at[w], src_v)
        pltpu.sync_copy(dst_hbm.at[w], dst_v)
        plsc.subcore_barrier()

        def body(j, _):
            pltpu.async_copy(table_hbm.at[src_v.at[j]], rows_v, sem).wait()
            pltpu.sync_copy(rows_v, acc.at[dst_v.at[j]], add=True)
            return 0
        lax.fori_loop(0, NCHUNK, body, 0)

        plsc.subcore_barrier()
        pltpu.sync_copy(acc.at[pl.ds(s * RPT, RPT)],
                        out_hbm.at[c, pl.ds(s * RPT, RPT)])
    return _agg


_agg64 = _make_agg(HID)
_agg128 = _make_agg(OUT_CH)


def _pre_body(degp_ref, x_ref, w_ref, h_ref, dis_ref):
    deg = degp_ref[0] + degp_ref[1] + 1.0
    dis = lax.rsqrt(deg)
    dis_ref[...] = dis
    h = jnp.dot(x_ref[...], w_ref[...], preferred_element_type=jnp.float32)
    h_ref[...] = h * dis


def _mid_body(sp_ref, hp_ref, dis_ref, b_ref, g_ref, be_ref, w_ref, out_ref):
    dis = dis_ref[...]
    z = dis * (sp_ref[0] + sp_ref[1] + hp_ref[...]) + b_ref[...]
    rows = lax.broadcasted_iota(jnp.int32, z.shape, 0)
    mask = rows < N
    zm = jnp.where(mask, z, 0.0)
    mean = jnp.sum(zm, axis=0, keepdims=True) / N
    var = jnp.sum(zm * zm, axis=0, keepdims=True) / N - mean * mean
    y = g_ref[...] * (z - mean) * lax.rsqrt(var + 1e-5) + be_ref[...]
    y = jnp.where(mask, jnp.maximum(y, 0.0), 0.0)
    out_ref[...] = jnp.dot(
        y, w_ref[...], preferred_element_type=jnp.float32) * dis


def _final_body(sp_ref, hp_ref, dis_ref, b_ref, batch_ref, out_ref):
    z = dis_ref[...] * (sp_ref[0] + sp_ref[1] + hp_ref[...]) + b_ref[...]
    gid = lax.broadcasted_iota(jnp.int32, (NG, NP), 0)
    onehot = (batch_ref[...] == gid).astype(jnp.float32)
    sums = jnp.dot(onehot, z, preferred_element_type=jnp.float32)
    cnt = jnp.sum(onehot, axis=1, keepdims=True)
    pooled = sums / jnp.maximum(cnt, 1.0)
    out_ref[...] = 1.0 / (1.0 + jnp.exp(-pooled))


def _f32(*shape):
    return jax.ShapeDtypeStruct(shape, jnp.float32)


def kernel(x, edge_index, batch, W0, b0, W1, b1, W2, b2, g0, be0, g1, be1):
    src = edge_index[0].astype(jnp.int32)
    dst = edge_index[1].astype(jnp.int32)
    pad = jnp.full((EP - E,), N, jnp.int32)
    src_r = jnp.concatenate([src, pad]).reshape(NC * NS, NCHUNK, CL)
    dst_r = jnp.concatenate([dst, pad]).reshape(NC * NS, NCHUNK, CL)
    x_p = jnp.pad(x, ((0, NP - N), (0, 0)))
    batch_p = jnp.concatenate(
        [batch.astype(jnp.int32),
         jnp.full((NP - N,), NG, jnp.int32)]).reshape(1, NP)

    degp = _deg_kernel(dst_r).reshape(NC, NP, 1)

    h0p, dis = pl.pallas_call(
        _pre_body, out_shape=(_f32(NP, HID), _f32(NP, 1)))(degp, x_p, W0)

    s0 = _agg64(h0p, src_r, dst_r)
    h1p = pl.pallas_call(_mid_body, out_shape=_f32(NP, HID))(
        s0, h0p, dis, b0.reshape(1, -1), g0.reshape(1, -1),
        be0.reshape(1, -1), W1)

    s1 = _agg64(h1p, src_r, dst_r)
    h2p = pl.pallas_call(_mid_body, out_shape=_f32(NP, OUT_CH))(
        s1, h1p, dis, b1.reshape(1, -1), g1.reshape(1, -1),
        be1.reshape(1, -1), W2)

    s2 = _agg128(h2p, src_r, dst_r)
    out = pl.pallas_call(_final_body, out_shape=_f32(NG, OUT_CH))(
        s2, h2p, dis, b2.reshape(1, -1), batch_p)
    return out
